# baseline (speedup 1.0000x reference)
import functools

import jax
import jax.numpy as jnp
from jax import lax
from jax.experimental import pallas as pl
from jax.experimental.pallas import tpu as pltpu
from jax.experimental.pallas import tpu_sc as plsc

N = 10000
NP = 10240
LN = 128
NTILES = 16
NCORES = 2
RPT = NP // NTILES
EG = 128
G = 96
GB = 24
NB = G // GB
E_CAP = NTILES * G * EG
BM = 512


@functools.lru_cache(maxsize=None)
def _make_prop(C):
    CH = (C + 1) // 2
    mesh = plsc.VectorSubcoreMesh(core_axis_name="c", subcore_axis_name="s")

    def body(hs, sidx, didx, out, acc, ra, rb,
             iv0, iv1, dv0, dv1, sema, semb, semi):
        cid = lax.axis_index("c")
        sid = lax.axis_index("s")
        zvec = jnp.zeros((16,), jnp.float32)
        ivs = (iv0, iv1)
        dvs = (dv0, dv1)

        for kc in range(CH):
            chunk = kc * NCORES + cid

            @pl.when(chunk < C)
            def _():
                def zrow(i, carry):
                    for j in range(8):
                        ra[i, pl.ds(j * 16, 16)] = zvec
                    return carry

                lax.fori_loop(0, 128, zrow, 0)
                for j in range(RPT // 128):
                    pltpu.sync_copy(ra, acc.at[pl.ds(sid * RPT + j * 128, 128)])
                plsc.subcore_barrier()

                pltpu.async_copy(sidx.at[chunk, sid, 0], iv0, semi)
                pltpu.async_copy(didx.at[sid, 0], dv0, semi)
                for nb in range(NB):
                    iv, dv = ivs[nb % 2], dvs[nb % 2]
                    pltpu.make_async_copy(sidx.at[chunk, sid, nb], iv, semi).wait()
                    pltpu.make_async_copy(didx.at[sid, nb], dv, semi).wait()
                    if nb + 1 < NB:
                        pltpu.async_copy(sidx.at[chunk, sid, nb + 1],
                                         ivs[(nb + 1) % 2], semi)
                        pltpu.async_copy(didx.at[sid, nb + 1],
                                         dvs[(nb + 1) % 2], semi)
                    pltpu.async_copy(hs.at[iv.at[0]], ra, sema)

                    def gbody(i, c2, iv=iv, dv=dv):
                        g = i * 2
                        pltpu.async_copy(hs.at[iv.at[g + 1]], rb, semb)
                        pltpu.make_async_copy(hs.at[iv.at[g]], ra, sema).wait()
                        pltpu.sync_copy(ra, acc.at[dv.at[g]], add=True)

                        @pl.when(g + 2 < GB)
                        def _():
                            pltpu.async_copy(hs.at[iv.at[g + 2]], ra, sema)

                        pltpu.make_async_copy(hs.at[iv.at[g + 1]], rb, semb).wait()
                        pltpu.sync_copy(rb, acc.at[dv.at[g + 1]], add=True)
                        return c2

                    lax.fori_loop(0, GB // 2, gbody, 0)
                plsc.subcore_barrier()
                pltpu.sync_copy(
                    acc.at[pl.ds(sid * RPT, RPT)],
                    out.at[pl.ds(chunk * NP + sid * RPT, RPT)])

    return pl.kernel(
        body,
        mesh=mesh,
        out_type=jax.ShapeDtypeStruct((C * NP, LN), jnp.float32),
        scratch_types=[
            pltpu.VMEM_SHARED((NP, LN), jnp.float32),
            pltpu.VMEM((EG, LN), jnp.float32),
            pltpu.VMEM((EG, LN), jnp.float32),
            pltpu.VMEM((GB, EG), jnp.int32),
            pltpu.VMEM((GB, EG), jnp.int32),
            pltpu.VMEM((GB, EG), jnp.int32),
            pltpu.VMEM((GB, EG), jnp.int32),
            pltpu.SemaphoreType.DMA,
            pltpu.SemaphoreType.DMA,
            pltpu.SemaphoreType.DMA,
        ],
    )


def _prop(C, hs3, sidx, didx):
    out = _make_prop(C)(hs3.reshape(C * NP, LN), sidx, didx)
    return out.reshape(C, NP, LN)


def _mm(x, w, b, d2, init, mode):
    if mode == "raw":
        K = x.shape[1]
    else:
        K = x.shape[0] * LN
    Fout = w.shape[1]
    Cin = K // LN
    BKC = 2 if Cin % 2 == 0 else 1
    KG = Cin // BKC
    Cout = Fout // LN
    w3 = w.reshape(Cin, LN, Fout)

    grid = (NP // BM, Cout, KG)

    if mode == "raw":
        x_spec = pl.BlockSpec((BM, BKC * LN), lambda i, j, k: (i, k))
    else:
        x_spec = pl.BlockSpec((BKC, BM, LN), lambda i, j, k: (k, i, 0))
    w_spec = pl.BlockSpec((BKC, LN, LN), lambda i, j, k: (k, 0, j))
    d_spec = pl.BlockSpec((BM, LN), lambda i, j, k: (i, 0))
    io_spec = pl.BlockSpec((1, BM, LN), lambda i, j, k: (j, i, 0))

    in_specs = [x_spec, w_spec, d_spec]
    args = [x, w3, d2]
    if mode == "mid":
        in_specs.append(pl.BlockSpec((BKC, 1, LN), lambda i, j, k: (k, 0, 0)))
        args.append(b.reshape(Cin, 1, LN))
    if init is not None:
        in_specs.append(io_spec)
        args.append(init)

    def body(*refs):
        if mode == "mid" and init is not None:
            x_ref, w_ref, d_ref, b_ref, i_ref, o_ref, acc = refs
        elif mode == "mid":
            x_ref, w_ref, d_ref, b_ref, o_ref, acc = refs
            i_ref = None
        elif init is not None:
            x_ref, w_ref, d_ref, i_ref, o_ref, acc = refs
        else:
            x_ref, w_ref, d_ref, o_ref, acc = refs
            i_ref = None
        k = pl.program_id(2)

        @pl.when(k == 0)
        def _():
            acc[...] = jnp.zeros((BM, LN), jnp.float32)

        d1 = d_ref[:, :1]
        if mode == "mid":
            xs = [jnp.maximum(d1 * x_ref[t] + b_ref[t, 0][None, :], 0.0)
                  for t in range(BKC)]
            xb = xs[0] if BKC == 1 else jnp.concatenate(xs, axis=1)
        else:
            xb = x_ref[...]
        wb = w_ref[0] if BKC == 1 else jnp.concatenate([w_ref[0], w_ref[1]], axis=0)
        acc[...] += jnp.dot(xb, wb, preferred_element_type=jnp.float32)

        @pl.when(k == KG - 1)
        def _():
            r = d1 * acc[...]
            if i_ref is not None:
                r = r + i_ref[0]
            o_ref[0] = r

    return pl.pallas_call(
        body,
        grid=grid,
        in_specs=in_specs,
        out_specs=io_spec,
        out_shape=jax.ShapeDtypeStruct((Cout, NP, LN), jnp.float32),
        scratch_shapes=[pltpu.VMEM((BM, LN), jnp.float32)],
        compiler_params=pltpu.CompilerParams(
            dimension_semantics=("parallel", "parallel", "arbitrary")),
    )(*args)


def _elemwise(body, out_shape, *arrays):
    spec = pl.BlockSpec((BM, LN), lambda i: (i, 0))
    return pl.pallas_call(
        body,
        grid=(NP // BM,),
        in_specs=[spec] * len(arrays),
        out_specs=spec,
        out_shape=out_shape,
    )(*arrays)


def _dinv2(sdeg, mask2):
    def body(s_ref, m_ref, o_ref):
        o_ref[...] = m_ref[...] * lax.rsqrt(jnp.maximum(s_ref[...], 1.0))

    return _elemwise(body, jax.ShapeDtypeStruct((NP, LN), jnp.float32), sdeg, mask2)


def _finalize(s, b2, d2):
    bfull = jnp.broadcast_to(b2[None, :], (NP, LN))

    def body(s_ref, b_ref, d_ref, o_ref):
        o_ref[...] = d_ref[...] * s_ref[...] + b_ref[...]

    return _elemwise(body, jax.ShapeDtypeStruct((NP, LN), jnp.float32),
                     s, bfull, d2)


def _pad_w(w, rows, cols):
    return jnp.pad(w, ((0, rows - w.shape[0]), (0, cols - w.shape[1])))


def kernel(vertices, feats1, feats2, feats3, edge_index, params):
    f32 = jnp.float32
    src = edge_index[0].astype(jnp.int32)
    dst = edge_index[1].astype(jnp.int32)
    sl = jnp.arange(N, dtype=jnp.int32)
    src_f = jnp.concatenate([src, sl])
    dst_f = jnp.concatenate([dst, sl])
    pad = E_CAP - src_f.shape[0]
    src_p = jnp.concatenate([src_f, jnp.full((pad,), NP - 1, jnp.int32)])
    dst_p = jnp.concatenate([dst_f, jnp.full((pad,), NP - 1, jnp.int32)])
    src_t = src_p.reshape(NTILES, NB, GB, EG)
    dst_t = dst_p.reshape(NTILES, NB, GB, EG)
    sidx = {c: src_t[None] + (jnp.arange(c, dtype=jnp.int32) * NP)[:, None, None, None, None]
            for c in (1, 2, 4, 8)}

    mask2 = jnp.broadcast_to(
        (jnp.arange(NP) < N).astype(f32)[:, None], (NP, LN))
    sdeg = _prop(1, mask2, sidx[1], dst_t)[0]
    d2 = _dinv2(sdeg, mask2)

    p1, p2, p3 = params["block1"], params["block2"], params["block3"]

    def chain_rest(hs0, p):
        s0 = _prop(8, hs0, sidx[8], dst_t)
        hs1 = _mm(s0, p["W1"], p["b0"], d2, None, "mid")
        s1 = _prop(4, hs1, sidx[4], dst_t)
        hs2 = _mm(s1, p["W2"], p["b1"], d2, None, "mid")
        s2 = _prop(2, hs2, sidx[2], dst_t)
        hs3 = _mm(s2, p["W3"], p["b2"], d2, None, "mid")
        s3 = _prop(1, hs3, sidx[1], dst_t)
        hs4 = _mm(s3, _pad_w(p["W4"], LN, LN), p["b3"], d2, None, "mid")
        s4 = _prop(1, hs4, sidx[1], dst_t)
        b4p = jnp.pad(p["b4"], (0, LN - 3))
        coord = _finalize(s4[0], b4p, d2)[:N, :3]
        return s3, coord

    x0 = jnp.concatenate([feats1, vertices], axis=1)
    x0 = jnp.pad(x0, ((0, NP - N), (0, 1536 - 1283)))
    hs0 = _mm(x0, _pad_w(p1["W0"], 1536, 1024), None, d2, None, "raw")
    s3_1, coord_1 = chain_rest(hs0, p1)

    pinit = _mm(s3_1, p2["W0"][1280:, :], p1["b3"], d2, None, "mid")
    f2p = jnp.pad(feats2, ((0, NP - N), (0, 0)))
    hs0 = _mm(f2p, p2["W0"][:1280, :], None, d2, pinit, "raw")
    s3_2, coord_2 = chain_rest(hs0, p2)

    pinit = _mm(s3_2, p3["W0"][1280:, :], p2["b3"], d2, None, "mid")
    f3p = jnp.pad(feats3, ((0, NP - N), (0, 0)))
    hs0 = _mm(f3p, p3["W0"][:1280, :], None, d2, pinit, "raw")
    _, coord_3 = chain_rest(hs0, p3)

    return (vertices, coord_1, coord_1, coord_2, coord_2, coord_3)


# speedup vs baseline: 1.0032x; 1.0032x over previous
import functools

import jax
import jax.numpy as jnp
from jax import lax
from jax.experimental import pallas as pl
from jax.experimental.pallas import tpu as pltpu
from jax.experimental.pallas import tpu_sc as plsc

N = 10000
NP = 10240
LN = 128
NTILES = 16
NCORES = 2
RPT = NP // NTILES
EG = 128
G = 96
GB = 8
NB = G // GB
E_CAP = NTILES * G * EG
BM = 512


@functools.lru_cache(maxsize=None)
def _make_prop(C):
    CH = (C + 1) // 2
    mesh = plsc.VectorSubcoreMesh(core_axis_name="c", subcore_axis_name="s")

    def body(hs, sidx, didx, out, acc, ra, rb,
             iv0, iv1, dv0, dv1, sema, semb, ssa, ssb, semi):
        cid = lax.axis_index("c")
        sid = lax.axis_index("s")
        zvec = jnp.zeros((16,), jnp.float32)
        ivs = (iv0, iv1)
        dvs = (dv0, dv1)
        ssems = (ssa, ssb)

        for kc in range(CH):
            chunk = kc * NCORES + cid

            @pl.when(chunk < C)
            def _():
                def zrow(i, carry):
                    for j in range(8):
                        ra[i, pl.ds(j * 16, 16)] = zvec
                    return carry

                lax.fori_loop(0, 128, zrow, 0)
                for j in range(RPT // 128):
                    pltpu.sync_copy(ra, acc.at[pl.ds(sid * RPT + j * 128, 128)])
                plsc.subcore_barrier()

                pltpu.async_copy(sidx.at[chunk, sid, 0], iv0, semi)
                pltpu.async_copy(didx.at[sid, 0], dv0, semi)
                bufs = (ra, rb)
                gsems = (sema, semb)
                for nb in range(NB):
                    iv, dv = ivs[nb % 2], dvs[nb % 2]
                    pltpu.make_async_copy(sidx.at[chunk, sid, nb], iv, semi).wait()
                    pltpu.make_async_copy(didx.at[sid, nb], dv, semi).wait()
                    if nb + 1 < NB:
                        pltpu.async_copy(sidx.at[chunk, sid, nb + 1],
                                         ivs[(nb + 1) % 2], semi)
                        pltpu.async_copy(didx.at[sid, nb + 1],
                                         dvs[(nb + 1) % 2], semi)

                    def gbody(i, c2, iv=iv, dv=dv, first=(nb == 0)):
                        cps = []
                        for t in range(2):
                            g = i * 2 + t
                            buf, gsem = bufs[t], gsems[t]

                            def swait(g=g, buf=buf, t=t):
                                pltpu.make_async_copy(
                                    buf, acc.at[dv.at[g]], ssems[t]).wait()

                            if first:
                                pl.when(i > 0)(swait)
                            else:
                                swait()
                            cps.append([
                                pltpu.async_copy(
                                    hs.at[iv.at[g, pl.ds(h * 64, 64)]],
                                    buf.at[pl.ds(h * 64, 64)], gsem)
                                for h in range(2)])
                        for t in range(2):
                            g = i * 2 + t
                            for cp in cps[t]:
                                cp.wait()
                            pltpu.async_copy(bufs[t], acc.at[dv.at[g]],
                                             ssems[t], add=True)
                        return c2

                    lax.fori_loop(0, GB // 2, gbody, 0)
                dvl = dvs[(NB - 1) % 2]
                for t in range(2):
                    pltpu.make_async_copy(bufs[t], acc.at[dvl.at[GB - 2 + t]],
                                          ssems[t]).wait()
                plsc.subcore_barrier()
                pltpu.sync_copy(
                    acc.at[pl.ds(sid * RPT, RPT)],
                    out.at[pl.ds(chunk * NP + sid * RPT, RPT)])

    return pl.kernel(
        body,
        mesh=mesh,
        out_type=jax.ShapeDtypeStruct((C * NP, LN), jnp.float32),
        scratch_types=[
            pltpu.VMEM_SHARED((NP, LN), jnp.float32),
            pltpu.VMEM((EG, LN), jnp.float32),
            pltpu.VMEM((EG, LN), jnp.float32),
            pltpu.VMEM((GB, EG), jnp.int32),
            pltpu.VMEM((GB, EG), jnp.int32),
            pltpu.VMEM((GB, EG), jnp.int32),
            pltpu.VMEM((GB, EG), jnp.int32),
            pltpu.SemaphoreType.DMA,
            pltpu.SemaphoreType.DMA,
            pltpu.SemaphoreType.DMA,
            pltpu.SemaphoreType.DMA,
            pltpu.SemaphoreType.DMA,
        ],
    )


def _prop(C, hs3, sidx, didx):
    out = _make_prop(C)(hs3.reshape(C * NP, LN), sidx, didx)
    return out.reshape(C, NP, LN)


def _mm(x, w, b, d2, init, mode):
    if mode == "raw":
        K = x.shape[1]
    else:
        K = x.shape[0] * LN
    Fout = w.shape[1]
    Cin = K // LN
    BKC = 2 if Cin % 2 == 0 else 1
    KG = Cin // BKC
    Cout = Fout // LN
    w3 = w.reshape(Cin, LN, Fout)

    grid = (NP // BM, Cout, KG)

    if mode == "raw":
        x_spec = pl.BlockSpec((BM, BKC * LN), lambda i, j, k: (i, k))
    else:
        x_spec = pl.BlockSpec((BKC, BM, LN), lambda i, j, k: (k, i, 0))
    w_spec = pl.BlockSpec((BKC, LN, LN), lambda i, j, k: (k, 0, j))
    d_spec = pl.BlockSpec((BM, LN), lambda i, j, k: (i, 0))
    io_spec = pl.BlockSpec((1, BM, LN), lambda i, j, k: (j, i, 0))

    in_specs = [x_spec, w_spec, d_spec]
    args = [x, w3, d2]
    if mode == "mid":
        in_specs.append(pl.BlockSpec((BKC, 1, LN), lambda i, j, k: (k, 0, 0)))
        args.append(b.reshape(Cin, 1, LN))
    if init is not None:
        in_specs.append(io_spec)
        args.append(init)

    def body(*refs):
        if mode == "mid" and init is not None:
            x_ref, w_ref, d_ref, b_ref, i_ref, o_ref, acc = refs
        elif mode == "mid":
            x_ref, w_ref, d_ref, b_ref, o_ref, acc = refs
            i_ref = None
        elif init is not None:
            x_ref, w_ref, d_ref, i_ref, o_ref, acc = refs
        else:
            x_ref, w_ref, d_ref, o_ref, acc = refs
            i_ref = None
        k = pl.program_id(2)

        @pl.when(k == 0)
        def _():
            acc[...] = jnp.zeros((BM, LN), jnp.float32)

        d1 = d_ref[:, :1]
        if mode == "mid":
            xs = [jnp.maximum(d1 * x_ref[t] + b_ref[t, 0][None, :], 0.0)
                  for t in range(BKC)]
            xb = xs[0] if BKC == 1 else jnp.concatenate(xs, axis=1)
        else:
            xb = x_ref[...]
        wb = w_ref[0] if BKC == 1 else jnp.concatenate([w_ref[0], w_ref[1]], axis=0)
        acc[...] += jnp.dot(xb, wb, preferred_element_type=jnp.float32)

        @pl.when(k == KG - 1)
        def _():
            r = d1 * acc[...]
            if i_ref is not None:
                r = r + i_ref[0]
            o_ref[0] = r

    return pl.pallas_call(
        body,
        grid=grid,
        in_specs=in_specs,
        out_specs=io_spec,
        out_shape=jax.ShapeDtypeStruct((Cout, NP, LN), jnp.float32),
        scratch_shapes=[pltpu.VMEM((BM, LN), jnp.float32)],
        compiler_params=pltpu.CompilerParams(
            dimension_semantics=("parallel", "parallel", "arbitrary")),
    )(*args)


def _elemwise(body, out_shape, *arrays):
    spec = pl.BlockSpec((BM, LN), lambda i: (i, 0))
    return pl.pallas_call(
        body,
        grid=(NP // BM,),
        in_specs=[spec] * len(arrays),
        out_specs=spec,
        out_shape=out_shape,
    )(*arrays)


def _dinv2(sdeg, mask2):
    def body(s_ref, m_ref, o_ref):
        o_ref[...] = m_ref[...] * lax.rsqrt(jnp.maximum(s_ref[...], 1.0))

    return _elemwise(body, jax.ShapeDtypeStruct((NP, LN), jnp.float32), sdeg, mask2)


def _finalize(s, b2, d2):
    bfull = jnp.broadcast_to(b2[None, :], (NP, LN))

    def body(s_ref, b_ref, d_ref, o_ref):
        o_ref[...] = d_ref[...] * s_ref[...] + b_ref[...]

    return _elemwise(body, jax.ShapeDtypeStruct((NP, LN), jnp.float32),
                     s, bfull, d2)


def _pad_w(w, rows, cols):
    return jnp.pad(w, ((0, rows - w.shape[0]), (0, cols - w.shape[1])))


def kernel(vertices, feats1, feats2, feats3, edge_index, params):
    f32 = jnp.float32
    src = edge_index[0].astype(jnp.int32)
    dst = edge_index[1].astype(jnp.int32)
    sl = jnp.arange(N, dtype=jnp.int32)
    src_f = jnp.concatenate([src, sl])
    dst_f = jnp.concatenate([dst, sl])
    pad = E_CAP - src_f.shape[0]
    src_p = jnp.concatenate([src_f, jnp.full((pad,), NP - 1, jnp.int32)])
    dst_p = jnp.concatenate([dst_f, jnp.full((pad,), NP - 1, jnp.int32)])
    src_t = src_p.reshape(NTILES, NB, GB, EG)
    dst_t = dst_p.reshape(NTILES, NB, GB, EG)
    sidx = {c: src_t[None] + (jnp.arange(c, dtype=jnp.int32) * NP)[:, None, None, None, None]
            for c in (1, 2, 4, 8)}

    mask2 = jnp.broadcast_to(
        (jnp.arange(NP) < N).astype(f32)[:, None], (NP, LN))
    sdeg = _prop(1, mask2, sidx[1], dst_t)[0]
    d2 = _dinv2(sdeg, mask2)

    p1, p2, p3 = params["block1"], params["block2"], params["block3"]

    def chain_rest(hs0, p):
        s0 = _prop(8, hs0, sidx[8], dst_t)
        hs1 = _mm(s0, p["W1"], p["b0"], d2, None, "mid")
        s1 = _prop(4, hs1, sidx[4], dst_t)
        hs2 = _mm(s1, p["W2"], p["b1"], d2, None, "mid")
        s2 = _prop(2, hs2, sidx[2], dst_t)
        hs3 = _mm(s2, p["W3"], p["b2"], d2, None, "mid")
        s3 = _prop(1, hs3, sidx[1], dst_t)
        hs4 = _mm(s3, _pad_w(p["W4"], LN, LN), p["b3"], d2, None, "mid")
        s4 = _prop(1, hs4, sidx[1], dst_t)
        b4p = jnp.pad(p["b4"], (0, LN - 3))
        coord = _finalize(s4[0], b4p, d2)[:N, :3]
        return s3, coord

    x0 = jnp.concatenate([feats1, vertices], axis=1)
    x0 = jnp.pad(x0, ((0, NP - N), (0, 1536 - 1283)))
    hs0 = _mm(x0, _pad_w(p1["W0"], 1536, 1024), None, d2, None, "raw")
    s3_1, coord_1 = chain_rest(hs0, p1)

    pinit = _mm(s3_1, p2["W0"][1280:, :], p1["b3"], d2, None, "mid")
    f2p = jnp.pad(feats2, ((0, NP - N), (0, 0)))
    hs0 = _mm(f2p, p2["W0"][:1280, :], None, d2, pinit, "raw")
    s3_2, coord_2 = chain_rest(hs0, p2)

    pinit = _mm(s3_2, p3["W0"][1280:, :], p2["b3"], d2, None, "mid")
    f3p = jnp.pad(feats3, ((0, NP - N), (0, 0)))
    hs0 = _mm(f3p, p3["W0"][:1280, :], None, d2, pinit, "raw")
    _, coord_3 = chain_rest(hs0, p3)

    return (vertices, coord_1, coord_1, coord_2, coord_2, coord_3)
